# baseline (speedup 1.0000x reference)
import jax
import jax.numpy as jnp
from jax import lax
from jax.experimental import pallas as pl
from jax.experimental.pallas import tpu as pltpu
from jax.experimental.pallas import tpu_sc as plsc

PRE_SEQ_LEN = 128
HIDDEN = 1024
NUM_LAYERS = 24
OUT_DIM = NUM_LAYERS * 2 * HIDDEN
BATCH = 4

NB = BATCH * PRE_SEQ_LEN
SPLIT = 24
DC = OUT_DIM // SPLIT
NBUF = 3
CPY = 128

NC, NS, L = 2, 16, 16
NW = NC * NS
B_PER_W = NB // NW


def _body(table, sidx_hbm, perm_hbm, out, idx_v, perm_v, buf, *sems):
    wid = lax.axis_index("s") * NC + lax.axis_index("c")
    base = wid * B_PER_W
    gsems = sems[:NBUF]
    wsems = sems[NBUF:]

    pltpu.sync_copy(sidx_hbm.at[pl.ds(base, B_PER_W)], idx_v)
    pltpu.sync_copy(perm_hbm.at[pl.ds(base, B_PER_W)], perm_v)
    sv = idx_v[...]

    lane = lax.iota(jnp.int32, L)
    s = [jnp.sum(jnp.where(lane == j, sv, 0)) for j in range(B_PER_W)]
    f = [None] + [s[j] != s[j - 1] for j in range(1, B_PER_W)]

    def gsrc(c, j):
        return table.at[pl.ds(s[j], 1), pl.ds(c * DC, DC)]

    def issue_gathers(c, i):
        pltpu.async_copy(gsrc(c, 0), buf.at[i, pl.ds(0, 1)], gsems[i])
        for j in range(1, B_PER_W):
            @pl.when(f[j])
            def _(c=c, i=i, j=j):
                pltpu.async_copy(gsrc(c, j), buf.at[i, pl.ds(j, 1)], gsems[i])

    def drain_gathers(c, i):
        pltpu.make_async_copy(gsrc(c, 0), buf.at[i, pl.ds(0, 1)], gsems[i]).wait()
        for j in range(1, B_PER_W):
            @pl.when(f[j])
            def _(c=c, i=i, j=j):
                pltpu.make_async_copy(
                    gsrc(c, j), buf.at[i, pl.ds(j, 1)], gsems[i]
                ).wait()

    def expand_dupes(i):
        for j in range(1, B_PER_W):
            @pl.when(jnp.logical_not(f[j]))
            def _(i=i, j=j):
                def cp(k, carry):
                    for m in range(CPY // L):
                        off = k * CPY + m * L
                        buf[i, j, pl.ds(off, L)] = buf[i, j - 1, pl.ds(off, L)]
                    return carry
                lax.fori_loop(0, DC // CPY, cp, jnp.int32(0))

    def issue_write(c, i):
        pltpu.async_copy(
            buf.at[i], out.at[perm_v, pl.ds(c * DC, DC)], wsems[i]
        )

    def drain_write(c, i):
        pltpu.make_async_copy(
            buf.at[i], out.at[perm_v, pl.ds(c * DC, DC)], wsems[i]
        ).wait()

    def q_body(q, carry):
        c0 = q * NBUF
        for i in range(NBUF):
            c = c0 + i

            @pl.when(q >= 1)
            def _(c=c, i=i):
                drain_write(c - NBUF, i)

            issue_gathers(c, i)

            def _tail(c=c, i=i):
                i1 = (i - 1) % NBUF
                drain_gathers(c - 1, i1)
                expand_dupes(i1)
                issue_write(c - 1, i1)

            if i == 0:
                pl.when(q >= 1)(_tail)
            else:
                _tail()
        return carry

    lax.fori_loop(0, SPLIT // NBUF, q_body, jnp.int32(0))

    last = SPLIT - 1
    li = last % NBUF
    drain_gathers(last, li)
    expand_dupes(li)
    issue_write(last, li)
    for c in range(SPLIT - NBUF, SPLIT):
        drain_write(c, c % NBUF)


@jax.jit
def _sc_gather(table, sidx, perm):
    mesh = plsc.VectorSubcoreMesh(core_axis_name="c", subcore_axis_name="s")
    k = pl.kernel(
        _body,
        out_type=jax.ShapeDtypeStruct((NB, OUT_DIM), jnp.float32),
        mesh=mesh,
        compiler_params=pltpu.CompilerParams(needs_layout_passes=False),
        scratch_types=(
            [pltpu.VMEM((B_PER_W,), jnp.int32)] * 2
            + [pltpu.VMEM((NBUF, B_PER_W, DC), jnp.float32)]
            + [pltpu.SemaphoreType.DMA] * (2 * NBUF)
        ),
    )
    return k(table, sidx, perm)


def kernel(prefix, embedding_weight):
    idx = prefix.reshape(NB)
    pos = lax.iota(jnp.int32, NB)
    sidx, perm = lax.sort_key_val(idx, pos)
    out = _sc_gather(embedding_weight, sidx, perm)
    return out.reshape(BATCH, PRE_SEQ_LEN, OUT_DIM)


# speedup vs baseline: 1.1808x; 1.1808x over previous
import jax
import jax.numpy as jnp
from jax import lax
from jax.experimental import pallas as pl
from jax.experimental.pallas import tpu as pltpu
from jax.experimental.pallas import tpu_sc as plsc

PRE_SEQ_LEN = 128
HIDDEN = 1024
NUM_LAYERS = 24
OUT_DIM = NUM_LAYERS * 2 * HIDDEN
BATCH = 4

NB = BATCH * PRE_SEQ_LEN
NC, NS, L = 2, 16, 16
NW = NC * NS
B_PER_W = NB // NW


def _body(table, sidx_hbm, perm_hbm, out, idx_v, perm_v, buf, gsem, wsem):
    wid = lax.axis_index("s") * NC + lax.axis_index("c")
    base = wid * B_PER_W

    pltpu.sync_copy(sidx_hbm.at[pl.ds(base, B_PER_W)], idx_v)
    pltpu.sync_copy(perm_hbm.at[pl.ds(base, B_PER_W)], perm_v)
    sv = idx_v[...]
    pv = perm_v[...]

    lane = lax.iota(jnp.int32, L)
    s = [jnp.sum(jnp.where(lane == j, sv, 0)) for j in range(B_PER_W)]
    p = [jnp.sum(jnp.where(lane == j, pv, 0)) for j in range(B_PER_W)]
    f = [None] + [s[j] != s[j - 1] for j in range(1, B_PER_W)]

    def wcopy(j):
        return pltpu.make_async_copy(buf, out.at[pl.ds(p[j], 1)], wsem)

    pltpu.async_copy(table.at[pl.ds(s[0], 1)], buf, gsem).wait()
    wcopy(0).start()

    for j in range(1, B_PER_W):
        wcopy(j - 1).wait()

        @pl.when(f[j])
        def _(j=j):
            pltpu.async_copy(table.at[pl.ds(s[j], 1)], buf, gsem).wait()

        wcopy(j).start()

    wcopy(B_PER_W - 1).wait()


@jax.jit
def _sc_gather(table, sidx, perm):
    mesh = plsc.VectorSubcoreMesh(core_axis_name="c", subcore_axis_name="s")
    k = pl.kernel(
        _body,
        out_type=jax.ShapeDtypeStruct((NB, OUT_DIM), jnp.float32),
        mesh=mesh,
        compiler_params=pltpu.CompilerParams(needs_layout_passes=False),
        scratch_types=[
            pltpu.VMEM((B_PER_W,), jnp.int32),
            pltpu.VMEM((B_PER_W,), jnp.int32),
            pltpu.VMEM((1, OUT_DIM), jnp.float32),
            pltpu.SemaphoreType.DMA,
            pltpu.SemaphoreType.DMA,
        ],
    )
    return k(table, sidx, perm)


def kernel(prefix, embedding_weight):
    idx = prefix.reshape(NB)
    pos = lax.iota(jnp.int32, NB)
    sidx, perm = lax.sort_key_val(idx, pos)
    out = _sc_gather(embedding_weight, sidx, perm)
    return out.reshape(BATCH, PRE_SEQ_LEN, OUT_DIM)


# speedup vs baseline: 1.2701x; 1.0756x over previous
import jax
import jax.numpy as jnp
from jax import lax
from jax.experimental import pallas as pl
from jax.experimental.pallas import tpu as pltpu
from jax.experimental.pallas import tpu_sc as plsc

PRE_SEQ_LEN = 128
HIDDEN = 1024
NUM_LAYERS = 24
OUT_DIM = NUM_LAYERS * 2 * HIDDEN
BATCH = 4

NB = BATCH * PRE_SEQ_LEN
NC, NS, L = 2, 16, 16
NW = NC * NS
B_PER_W = NB // NW


def _body(table, sidx_hbm, perm_hbm, out, idx_v, perm_v, buf_a, buf_b,
          gsem, *wsems):
    wid = lax.axis_index("s") * NC + lax.axis_index("c")
    base = wid * B_PER_W

    pltpu.sync_copy(sidx_hbm.at[pl.ds(base, B_PER_W)], idx_v)
    pltpu.sync_copy(perm_hbm.at[pl.ds(base, B_PER_W)], perm_v)
    sv = idx_v[...]
    pv = perm_v[...]

    lane = lax.iota(jnp.int32, L)
    s = [jnp.sum(jnp.where(lane == j, sv, 0)) for j in range(B_PER_W)]
    p = [jnp.sum(jnp.where(lane == j, pv, 0)) for j in range(B_PER_W)]
    f = [None] + [s[j] != s[j - 1] for j in range(1, B_PER_W)]
    r = [jnp.int32(0)]
    for j in range(1, B_PER_W):
        r.append(r[j - 1] + jnp.where(f[j], 1, 0).astype(jnp.int32))
    q = [jnp.equal(lax.rem(r[j], 2), 0) for j in range(B_PER_W)]

    def drain(j):
        pltpu.make_async_copy(buf_a, out.at[pl.ds(p[j], 1)], wsems[j]).wait()

    for j in range(B_PER_W):
        if j >= 2:
            drain(j - 2)

        if j == 0:
            pltpu.async_copy(table.at[pl.ds(s[0], 1)], buf_a, gsem).wait()
        else:
            @pl.when(f[j] & q[j])
            def _(j=j):
                pltpu.async_copy(table.at[pl.ds(s[j], 1)], buf_a, gsem).wait()

            @pl.when(f[j] & jnp.logical_not(q[j]))
            def _(j=j):
                pltpu.async_copy(table.at[pl.ds(s[j], 1)], buf_b, gsem).wait()

        @pl.when(q[j])
        def _(j=j):
            pltpu.make_async_copy(buf_a, out.at[pl.ds(p[j], 1)], wsems[j]).start()

        @pl.when(jnp.logical_not(q[j]))
        def _(j=j):
            pltpu.make_async_copy(buf_b, out.at[pl.ds(p[j], 1)], wsems[j]).start()

    drain(B_PER_W - 2)
    drain(B_PER_W - 1)


@jax.jit
def _sc_gather(table, sidx, perm):
    mesh = plsc.VectorSubcoreMesh(core_axis_name="c", subcore_axis_name="s")
    k = pl.kernel(
        _body,
        out_type=jax.ShapeDtypeStruct((NB, OUT_DIM), jnp.float32),
        mesh=mesh,
        compiler_params=pltpu.CompilerParams(needs_layout_passes=False),
        scratch_types=(
            [pltpu.VMEM((B_PER_W,), jnp.int32)] * 2
            + [pltpu.VMEM((1, OUT_DIM), jnp.float32)] * 2
            + [pltpu.SemaphoreType.DMA] * (1 + B_PER_W)
        ),
    )
    return k(table, sidx, perm)


def kernel(prefix, embedding_weight):
    idx = prefix.reshape(NB)
    pos = lax.iota(jnp.int32, NB)
    sidx, perm = lax.sort_key_val(idx, pos)
    out = _sc_gather(embedding_weight, sidx, perm)
    return out.reshape(BATCH, PRE_SEQ_LEN, OUT_DIM)
